# Initial kernel scaffold; baseline (speedup 1.0000x reference)
import jax
import jax.numpy as jnp
from jax.experimental import pallas as pl


def kernel(x, edge_index, batch, Wp, bp, hidden_c, W1, b1, W2, b2, W_ih, W_hh, b_ih, b_hh, Wo, bo):
    raise NotImplementedError("write your pallas kernel here")


# speedup vs baseline: 11.2832x; 11.2832x over previous
import functools

import jax
import jax.numpy as jnp
from jax import lax
from jax.experimental import pallas as pl
from jax.experimental.pallas import tpu as pltpu
from jax.experimental.pallas import tpu_sc as plsc

_N = 10000
_H = 128
_SC_CORES = 2
_SC_TILES = 16
_NW = _SC_CORES * _SC_TILES
_CH = 128
_CHUNKS = 81
_EPT = _CH * _CHUNKS
_EPAD = _NW * _EPT
_ACC_ROWS = 10240
_ACC_SLAB = _ACC_ROWS // _SC_TILES
_OUT_SLAB = _N // _SC_TILES

_sc_mesh = plsc.VectorSubcoreMesh(core_axis_name="c", subcore_axis_name="s")


@functools.partial(
    pl.kernel,
    out_type=jax.ShapeDtypeStruct((_SC_CORES, _ACC_ROWS, 16), jnp.float32),
    mesh=_sc_mesh,
    scratch_types=[
        pltpu.VMEM((_CHUNKS, _CH), jnp.int32),
        pltpu.VMEM((_CH, 16), jnp.float32),
        pltpu.VMEM_SHARED((_ACC_ROWS, 16), jnp.float32),
    ],
)
def _sc_deg(dst_hbm, ones_hbm, zeros_hbm, out_hbm, dst_v, ones_v, acc):
    cid = lax.axis_index("c")
    sid = lax.axis_index("s")
    wid = cid * _SC_TILES + sid

    pltpu.sync_copy(zeros_hbm, acc.at[pl.ds(sid * _ACC_SLAB, _ACC_SLAB)])
    pltpu.sync_copy(dst_hbm.at[wid], dst_v)
    pltpu.sync_copy(ones_hbm, ones_v)
    plsc.subcore_barrier()

    def body(j, carry):
        pltpu.sync_copy(ones_v, acc.at[dst_v.at[j]], add=True)
        return carry

    lax.fori_loop(0, _CHUNKS, body, 0)
    plsc.subcore_barrier()
    pltpu.sync_copy(
        acc.at[pl.ds(sid * _ACC_SLAB, _ACC_SLAB)],
        out_hbm.at[cid, pl.ds(sid * _ACC_SLAB, _ACC_SLAB)],
    )


@functools.partial(
    pl.kernel,
    out_type=jax.ShapeDtypeStruct((_SC_CORES, _ACC_ROWS, _H), jnp.float32),
    mesh=_sc_mesh,
    scratch_types=[
        pltpu.VMEM((_CHUNKS, _CH), jnp.int32),
        pltpu.VMEM((_CHUNKS, _CH), jnp.int32),
        pltpu.VMEM((_CH, _H), jnp.float32),
        pltpu.VMEM_SHARED((_ACC_ROWS, _H), jnp.float32),
        pltpu.SemaphoreType.DMA,
    ],
)
def _sc_spmm(p_hbm, src_hbm, dst_hbm, zeros_hbm, out_hbm, src_v, dst_v, rows_v, acc, sem):
    cid = lax.axis_index("c")
    sid = lax.axis_index("s")
    wid = cid * _SC_TILES + sid

    pltpu.sync_copy(zeros_hbm, acc.at[pl.ds(sid * _ACC_SLAB, _ACC_SLAB)])
    pltpu.sync_copy(src_hbm.at[wid], src_v)
    pltpu.sync_copy(dst_hbm.at[wid], dst_v)
    plsc.subcore_barrier()

    def body(j, carry):
        pltpu.async_copy(p_hbm.at[src_v.at[j]], rows_v, sem).wait()
        pltpu.sync_copy(rows_v, acc.at[dst_v.at[j]], add=True)
        return carry

    lax.fori_loop(0, _CHUNKS, body, 0)
    plsc.subcore_barrier()
    pltpu.sync_copy(
        acc.at[pl.ds(sid * _ACC_SLAB, _ACC_SLAB)],
        out_hbm.at[cid, pl.ds(sid * _ACC_SLAB, _ACC_SLAB)],
    )


_BN = 1000
_GRID = _N // _BN


def _dinv_block(dp):
    deg = dp[0][:, 0:1] + dp[1][:, 0:1]
    return jnp.where(deg > 0.0, lax.rsqrt(deg), 0.0)


def _gru_math(gi, gh, h):
    r = jax.nn.sigmoid(gi[:, 0:_H] + gh[:, 0:_H])
    z = jax.nn.sigmoid(gi[:, _H:2 * _H] + gh[:, _H:2 * _H])
    n = jnp.tanh(gi[:, 2 * _H:] + r * gh[:, 2 * _H:])
    return (1.0 - z) * n + z * h


def _dot(a, b):
    return jnp.dot(a, b, preferred_element_type=jnp.float32)


def _row_spec():
    return pl.BlockSpec((_BN, _H), lambda i: (i, 0))


def _dp_spec():
    return pl.BlockSpec((2, _BN, 16), lambda i: (0, i, 0))


def _full_spec(shape):
    return pl.BlockSpec(shape, lambda i: tuple(0 for _ in shape))


def _tc_init_body(x_ref, dp_ref, hc_ref, wpt_ref, bp_ref, wiht_ref, whht_ref,
                  bih_ref, bhh_ref, w1t_ref, h_out, p_out):
    xb = x_ref[...]
    t = _dot(xb, wpt_ref[...]) + bp_ref[...]
    gi = _dot(t, wiht_ref[...]) + bih_ref[...]
    hc = hc_ref[...]
    gh_row = _dot(hc, whht_ref[...]) + bhh_ref[...]
    h0 = jnp.broadcast_to(hc, (xb.shape[0], _H))
    gh = jnp.broadcast_to(gh_row, (xb.shape[0], 3 * _H))
    hnew = _gru_math(gi, gh, h0)
    dinv = _dinv_block(dp_ref[...])
    h_out[...] = hnew
    p_out[...] = _dot(hnew, w1t_ref[...]) * dinv


def _tc_mid_body(s_ref, dp_ref, b1_ref, w2t_ref, p_out):
    s = s_ref[0] + s_ref[1]
    dinv = _dinv_block(dp_ref[...])
    c1 = jnp.maximum(s * dinv + b1_ref[...], 0.0)
    p_out[...] = _dot(c1, w2t_ref[...]) * dinv


def _tc_gru_body(s_ref, dp_ref, b2_ref, h_ref, wiht_ref, whht_ref,
                 bih_ref, bhh_ref, w1t_ref, h_out, p_out):
    s = s_ref[0] + s_ref[1]
    dinv = _dinv_block(dp_ref[...])
    c2 = jnp.maximum(s * dinv + b2_ref[...], 0.0)
    gi = _dot(c2, wiht_ref[...]) + bih_ref[...]
    h = h_ref[...]
    gh = _dot(h, whht_ref[...]) + bhh_ref[...]
    hnew = _gru_math(gi, gh, h)
    h_out[...] = hnew
    p_out[...] = _dot(hnew, w1t_ref[...]) * dinv


def _tc_out_body(h_ref, wot_ref, bo_ref, o_out):
    l = _dot(h_ref[...], wot_ref[...]) + bo_ref[...]
    m = jnp.max(l, axis=1, keepdims=True)
    e = jnp.exp(l - m)
    o_out[...] = e / jnp.sum(e, axis=1, keepdims=True)


def _tc_init(x, dp, hc, wpt, bp, wiht, whht, bih, bhh, w1t):
    return pl.pallas_call(
        _tc_init_body,
        grid=(_GRID,),
        in_specs=[
            _row_spec(), _dp_spec(), _full_spec((1, _H)), _full_spec((_H, _H)),
            _full_spec((1, _H)), _full_spec((_H, 3 * _H)), _full_spec((_H, 3 * _H)),
            _full_spec((1, 3 * _H)), _full_spec((1, 3 * _H)), _full_spec((_H, _H)),
        ],
        out_specs=[_row_spec(), _row_spec()],
        out_shape=[
            jax.ShapeDtypeStruct((_N, _H), jnp.float32),
            jax.ShapeDtypeStruct((_N, _H), jnp.float32),
        ],
    )(x, dp, hc, wpt, bp, wiht, whht, bih, bhh, w1t)


def _s_spec():
    return pl.BlockSpec((2, _BN, _H), lambda i: (0, i, 0))


def _tc_mid(s, dp, b1, w2t):
    return pl.pallas_call(
        _tc_mid_body,
        grid=(_GRID,),
        in_specs=[_s_spec(), _dp_spec(), _full_spec((1, _H)), _full_spec((_H, _H))],
        out_specs=[_row_spec()],
        out_shape=[jax.ShapeDtypeStruct((_N, _H), jnp.float32)],
    )(s, dp, b1, w2t)[0]


def _tc_gru(s, dp, b2, h, wiht, whht, bih, bhh, w1t):
    return pl.pallas_call(
        _tc_gru_body,
        grid=(_GRID,),
        in_specs=[
            _s_spec(), _dp_spec(), _full_spec((1, _H)), _row_spec(),
            _full_spec((_H, 3 * _H)), _full_spec((_H, 3 * _H)),
            _full_spec((1, 3 * _H)), _full_spec((1, 3 * _H)), _full_spec((_H, _H)),
        ],
        out_specs=[_row_spec(), _row_spec()],
        out_shape=[
            jax.ShapeDtypeStruct((_N, _H), jnp.float32),
            jax.ShapeDtypeStruct((_N, _H), jnp.float32),
        ],
    )(s, dp, b2, h, wiht, whht, bih, bhh, w1t)


def _tc_out(h, wot, bo):
    return pl.pallas_call(
        _tc_out_body,
        grid=(_GRID,),
        in_specs=[_row_spec(), _full_spec((_H, _H)), _full_spec((1, _H))],
        out_specs=[_row_spec()],
        out_shape=[jax.ShapeDtypeStruct((_N, _H), jnp.float32)],
    )(h, wot, bo)[0]


def kernel(x, edge_index, batch, Wp, bp, hidden_c, W1, b1, W2, b2,
           W_ih, W_hh, b_ih, b_hh, Wo, bo):
    n = x.shape[0]
    e = edge_index.shape[1]
    pad = _EPAD - (e + n)
    loops = jnp.arange(n, dtype=jnp.int32)
    srcp = jnp.concatenate(
        [edge_index[0], loops, jnp.zeros((pad,), jnp.int32)]
    ).reshape(_NW, _CHUNKS, _CH)
    dstp = jnp.concatenate(
        [edge_index[1], loops, jnp.full((pad,), n, jnp.int32)]
    ).reshape(_NW, _CHUNKS, _CH)

    ones16 = jnp.ones((_CH, 16), jnp.float32)
    zeros16 = jnp.zeros((_ACC_SLAB, 16), jnp.float32)
    zerosH = jnp.zeros((_ACC_SLAB, _H), jnp.float32)

    dp = _sc_deg(dstp, ones16, zeros16)

    hc = hidden_c.reshape(1, _H)
    wpt, w1t, w2t = Wp.T, W1.T, W2.T
    wiht, whht = W_ih.T, W_hh.T
    bp2, b12, b22 = bp.reshape(1, -1), b1.reshape(1, -1), b2.reshape(1, -1)
    bih2, bhh2 = b_ih.reshape(1, -1), b_hh.reshape(1, -1)

    h, p = _tc_init(x, dp, hc, wpt, bp2, wiht, whht, bih2, bhh2, w1t)
    for _ in range(3):
        s1 = _sc_spmm(p, srcp, dstp, zerosH)
        p = _tc_mid(s1, dp, b12, w2t)
        s2 = _sc_spmm(p, srcp, dstp, zerosH)
        h, p = _tc_gru(s2, dp, b22, h, wiht, whht, bih2, bhh2, w1t)

    wot = jnp.zeros((_H, _H), jnp.float32).at[:, :Wo.shape[0]].set(Wo.T)
    bo2 = jnp.full((1, _H), -1e30, jnp.float32).at[0, :Wo.shape[0]].set(bo)
    probs = _tc_out(h, wot, bo2)
    return probs[:, :Wo.shape[0]]
